# baseline (speedup 1.0000x reference)
import jax
import jax.numpy as jnp
from jax import lax
from jax.experimental import pallas as pl
from jax.experimental.pallas import tpu as pltpu
from jax.experimental.pallas import tpu_sc as plsc

N = 10000
D = 128
E = 320000
NC = 2
NS = 16
L = 16
NW = NC * NS
CH = 80
EPT = E // NW
NSLOT = EPT // CH
NPAD = 10112
RPT = NPAD // NS
ZC = RPT // CH
ZREM = RPT - ZC * CH


def _splat(wvec, k):
    return lax.gather(
        wvec, jnp.full((L, 1), k, jnp.int32),
        lax.GatherDimensionNumbers(
            offset_dims=(), collapsed_slice_dims=(0,), start_index_map=(0,)),
        slice_sizes=(1,),
        mode=lax.GatherScatterMode.PROMISE_IN_BOUNDS)


def _sc_spmm(x_hbm, src_hbm, dst_hbm, w_hbm, out_hbm,
             agg, dst_all, w_all,
             rows0, rows1, sv0, sv1, gsem, gsem2, ssem, isem0, isem1):
    rows = (rows0, rows1)
    svs = (sv0, sv1)
    isems = (isem0, isem1)
    c = lax.axis_index("c")
    s = lax.axis_index("s")
    wid = c * NS + s
    row0 = s * RPT
    ebase = wid * EPT

    pltpu.sync_copy(dst_hbm.at[pl.ds(ebase, EPT)], dst_all)
    pltpu.sync_copy(w_hbm.at[pl.ds(ebase, EPT)], w_all)

    def _zrow(r, carry):
        for j in range(D // L):
            rows0[r, pl.ds(j * L, L)] = jnp.zeros((L,), jnp.float32)
        return carry
    lax.fori_loop(0, CH, _zrow, 0)
    for k in range(ZC):
        pltpu.sync_copy(rows0, agg.at[pl.ds(row0 + k * CH, CH)])
    pltpu.sync_copy(rows0.at[pl.ds(0, ZREM)],
                    agg.at[pl.ds(row0 + ZC * CH, ZREM)])
    plsc.subcore_barrier()

    def _src_dma(b, g):
        return pltpu.make_async_copy(
            src_hbm.at[pl.ds(ebase + g * CH, CH)], svs[b], isems[b])

    H = CH // 2

    def _gather_lo(b):
        return pltpu.make_async_copy(
            x_hbm.at[svs[b].at[pl.ds(0, H)]], rows[b].at[pl.ds(0, H)], gsem)

    def _gather_hi(b):
        return pltpu.make_async_copy(
            x_hbm.at[svs[b].at[pl.ds(H, H)]], rows[b].at[pl.ds(H, H)], gsem2)

    def _gather_start(b):
        _gather_lo(b).start()
        _gather_hi(b).start()

    def _gather_wait(b):
        _gather_lo(b).wait()
        _gather_hi(b).wait()

    def _scatter(b, off):
        return pltpu.make_async_copy(
            rows[b], agg.at[dst_all.at[pl.ds(off, CH)]], ssem)

    def _scale(b, off):
        for blk in range(CH // L):
            wvec = w_all[pl.ds(off + blk * L, L)]
            for k in range(L):
                wsplat = _splat(wvec, k)
                e = blk * L + k
                for j in range(D // L):
                    sl = pl.ds(j * L, L)
                    rows[b][e, sl] = rows[b][e, sl] * wsplat

    _src_dma(0, 0).start()
    _src_dma(0, 0).wait()
    _gather_start(0)
    _src_dma(1, 1).start()

    def _slot(g, b):
        b1 = 1 - b
        _gather_wait(b)

        @pl.when(g < NSLOT - 2)
        def _():
            _src_dma(b, g + 2).start()

        @pl.when(g >= 1)
        def _():
            _scatter(b1, (g - 1) * CH).wait()

        @pl.when(g < NSLOT - 1)
        def _():
            _src_dma(b1, g + 1).wait()
            _gather_start(b1)
        _scale(b, g * CH)
        _scatter(b, g * CH).start(add=True)

    def _outer(t, carry):
        for j in range(2):
            g = t * 2 + j

            @pl.when(g < NSLOT)
            def _():
                _slot(g, j)
        return carry
    lax.fori_loop(0, (NSLOT + 1) // 2, _outer, 0)

    _scatter((NSLOT - 1) % 2, (NSLOT - 1) * CH).wait()
    plsc.subcore_barrier()
    last = N - (NS - 1) * RPT

    @pl.when(s < NS - 1)
    def _():
        pltpu.sync_copy(agg.at[pl.ds(row0, RPT)],
                        out_hbm.at[pl.ds(c * N + row0, RPT)])

    @pl.when(s == NS - 1)
    def _():
        pltpu.sync_copy(agg.at[pl.ds((NS - 1) * RPT, last)],
                        out_hbm.at[pl.ds(c * N + (NS - 1) * RPT, last)])


_sc_call = pl.kernel(
    _sc_spmm,
    out_type=jax.ShapeDtypeStruct((2 * N, D), jnp.float32),
    mesh=plsc.VectorSubcoreMesh(core_axis_name="c", subcore_axis_name="s",
                                num_cores=NC, num_subcores=NS),
    scratch_types=[
        pltpu.VMEM_SHARED((NPAD, D), jnp.float32),
        pltpu.VMEM((EPT,), jnp.int32),
        pltpu.VMEM((EPT,), jnp.float32),
        pltpu.VMEM((CH, D), jnp.float32),
        pltpu.VMEM((CH, D), jnp.float32),
        pltpu.VMEM((CH,), jnp.int32),
        pltpu.VMEM((CH,), jnp.int32),
        pltpu.SemaphoreType.DMA,
        pltpu.SemaphoreType.DMA,
        pltpu.SemaphoreType.DMA,
        pltpu.SemaphoreType.DMA,
        pltpu.SemaphoreType.DMA,
    ],
)

BR = 1000


def _tc_body(x_ref, p0_ref, p1_ref, w_ref, b_ref, o_ref):
    sm = p0_ref[...] + p1_ref[...]
    o_ref[...] = x_ref[...] + b_ref[...] + jnp.dot(
        sm, w_ref[...], preferred_element_type=jnp.float32,
        precision=lax.Precision.HIGHEST)


def _tc_call(x, p0, p1, W, b2):
    return pl.pallas_call(
        _tc_body,
        grid=(N // BR,),
        in_specs=[
            pl.BlockSpec((BR, D), lambda i: (i, 0)),
            pl.BlockSpec((BR, D), lambda i: (i, 0)),
            pl.BlockSpec((BR, D), lambda i: (i, 0)),
            pl.BlockSpec((D, D), lambda i: (0, 0)),
            pl.BlockSpec((1, D), lambda i: (0, 0)),
        ],
        out_specs=pl.BlockSpec((BR, D), lambda i: (i, 0)),
        out_shape=jax.ShapeDtypeStruct((N, D), jnp.float32),
    )(x, p0, p1, W, b2)


def kernel(x, edge_index, edge_weight, W, b):
    dst = edge_index[0]
    src = edge_index[1]
    parts = _sc_call(x, src, dst, edge_weight)
    embed = _tc_call(x, parts[:N], parts[N:], W, b.reshape(1, D))
    return (embed, 0)


# speedup vs baseline: 1.0072x; 1.0072x over previous
import jax
import jax.numpy as jnp
from jax import lax
from jax.experimental import pallas as pl
from jax.experimental.pallas import tpu as pltpu
from jax.experimental.pallas import tpu_sc as plsc

N = 10000
D = 128
E = 320000
NC = 2
NS = 16
L = 16
NW = NC * NS
CH = 80
EPT = E // NW
NSLOT = EPT // CH
NPAD = 10112
RPT = NPAD // NS
ZC = RPT // CH
ZREM = RPT - ZC * CH


def _splat(wvec, k):
    return lax.gather(
        wvec, jnp.full((L, 1), k, jnp.int32),
        lax.GatherDimensionNumbers(
            offset_dims=(), collapsed_slice_dims=(0,), start_index_map=(0,)),
        slice_sizes=(1,),
        mode=lax.GatherScatterMode.PROMISE_IN_BOUNDS)


def _sc_spmm(x_hbm, src_hbm, dst_hbm, w_hbm, out_hbm,
             agg, dst_all, w_all,
             rows0, rows1, sv0, sv1, gsem, gsem2, ssem, isem0, isem1):
    rows = (rows0, rows1)
    svs = (sv0, sv1)
    isems = (isem0, isem1)
    c = lax.axis_index("c")
    s = lax.axis_index("s")
    wid = c * NS + s
    row0 = s * RPT
    ebase = wid * EPT

    pltpu.sync_copy(dst_hbm.at[pl.ds(ebase, EPT)], dst_all)
    pltpu.sync_copy(w_hbm.at[pl.ds(ebase, EPT)], w_all)

    def _zrow(r, carry):
        for j in range(D // L):
            rows0[r, pl.ds(j * L, L)] = jnp.zeros((L,), jnp.float32)
        return carry
    lax.fori_loop(0, CH, _zrow, 0)
    for k in range(ZC):
        pltpu.sync_copy(rows0, agg.at[pl.ds(row0 + k * CH, CH)])
    pltpu.sync_copy(rows0.at[pl.ds(0, ZREM)],
                    agg.at[pl.ds(row0 + ZC * CH, ZREM)])
    plsc.subcore_barrier()

    def _src_dma(b, g):
        return pltpu.make_async_copy(
            src_hbm.at[pl.ds(ebase + g * CH, CH)], svs[b], isems[b])

    H = CH // 2

    def _gather_lo(b):
        return pltpu.make_async_copy(
            x_hbm.at[svs[b].at[pl.ds(0, H)]], rows[b].at[pl.ds(0, H)], gsem)

    def _gather_hi(b):
        return pltpu.make_async_copy(
            x_hbm.at[svs[b].at[pl.ds(H, H)]], rows[b].at[pl.ds(H, H)], gsem2)

    def _gather_start(b):
        _gather_lo(b).start()
        _gather_hi(b).start()

    def _gather_wait(b):
        _gather_lo(b).wait()
        _gather_hi(b).wait()

    def _scatter(b, off):
        return pltpu.make_async_copy(
            rows[b], agg.at[dst_all.at[pl.ds(off, CH)]], ssem)

    def _scale(b, off):
        def _blk(blk, carry):
            wvec = w_all[pl.ds(off + blk * L, L)]
            for k in range(L):
                wsplat = _splat(wvec, k)
                e = blk * L + k
                for j in range(D // L):
                    sl = pl.ds(j * L, L)
                    rows[b][e, sl] = rows[b][e, sl] * wsplat
            return carry
        lax.fori_loop(0, CH // L, _blk, 0)

    _src_dma(0, 0).start()
    _src_dma(0, 0).wait()
    _gather_start(0)
    _src_dma(1, 1).start()

    def _slot(g, b):
        b1 = 1 - b
        _gather_wait(b)

        @pl.when(g < NSLOT - 2)
        def _():
            _src_dma(b, g + 2).start()

        @pl.when(g >= 1)
        def _():
            _scatter(b1, (g - 1) * CH).wait()

        @pl.when(g < NSLOT - 1)
        def _():
            _src_dma(b1, g + 1).wait()
            _gather_start(b1)
        _scale(b, g * CH)
        _scatter(b, g * CH).start(add=True)

    def _outer(t, carry):
        for j in range(2):
            g = t * 2 + j

            @pl.when(g < NSLOT)
            def _():
                _slot(g, j)
        return carry
    lax.fori_loop(0, (NSLOT + 1) // 2, _outer, 0)

    _scatter((NSLOT - 1) % 2, (NSLOT - 1) * CH).wait()
    plsc.subcore_barrier()
    last = N - (NS - 1) * RPT

    @pl.when(s < NS - 1)
    def _():
        pltpu.sync_copy(agg.at[pl.ds(row0, RPT)],
                        out_hbm.at[pl.ds(c * N + row0, RPT)])

    @pl.when(s == NS - 1)
    def _():
        pltpu.sync_copy(agg.at[pl.ds((NS - 1) * RPT, last)],
                        out_hbm.at[pl.ds(c * N + (NS - 1) * RPT, last)])


_sc_call = pl.kernel(
    _sc_spmm,
    out_type=jax.ShapeDtypeStruct((2 * N, D), jnp.float32),
    mesh=plsc.VectorSubcoreMesh(core_axis_name="c", subcore_axis_name="s",
                                num_cores=NC, num_subcores=NS),
    scratch_types=[
        pltpu.VMEM_SHARED((NPAD, D), jnp.float32),
        pltpu.VMEM((EPT,), jnp.int32),
        pltpu.VMEM((EPT,), jnp.float32),
        pltpu.VMEM((CH, D), jnp.float32),
        pltpu.VMEM((CH, D), jnp.float32),
        pltpu.VMEM((CH,), jnp.int32),
        pltpu.VMEM((CH,), jnp.int32),
        pltpu.SemaphoreType.DMA,
        pltpu.SemaphoreType.DMA,
        pltpu.SemaphoreType.DMA,
        pltpu.SemaphoreType.DMA,
        pltpu.SemaphoreType.DMA,
    ],
)

BR = 1000


def _tc_body(x_ref, p0_ref, p1_ref, w_ref, b_ref, o_ref):
    sm = p0_ref[...] + p1_ref[...]
    o_ref[...] = x_ref[...] + b_ref[...] + jnp.dot(
        sm, w_ref[...], preferred_element_type=jnp.float32,
        precision=lax.Precision.HIGHEST)


def _tc_call(x, p0, p1, W, b2):
    return pl.pallas_call(
        _tc_body,
        grid=(N // BR,),
        in_specs=[
            pl.BlockSpec((BR, D), lambda i: (i, 0)),
            pl.BlockSpec((BR, D), lambda i: (i, 0)),
            pl.BlockSpec((BR, D), lambda i: (i, 0)),
            pl.BlockSpec((D, D), lambda i: (0, 0)),
            pl.BlockSpec((1, D), lambda i: (0, 0)),
        ],
        out_specs=pl.BlockSpec((BR, D), lambda i: (i, 0)),
        out_shape=jax.ShapeDtypeStruct((N, D), jnp.float32),
    )(x, p0, p1, W, b2)


def kernel(x, edge_index, edge_weight, W, b):
    dst = edge_index[0]
    src = edge_index[1]
    parts = _sc_call(x, src, dst, edge_weight)
    embed = _tc_call(x, parts[:N], parts[N:], W, b.reshape(1, D))
    return (embed, 0)


# speedup vs baseline: 1.0121x; 1.0048x over previous
import jax
import jax.numpy as jnp
from jax import lax
from jax.experimental import pallas as pl
from jax.experimental.pallas import tpu as pltpu
from jax.experimental.pallas import tpu_sc as plsc

N = 10000
D = 128
E = 320000
NC = 2
NS = 16
L = 16
NW = NC * NS
CH = 80
EPT = E // NW
NSLOT = EPT // CH
NPAD = 10112
RPT = NPAD // NS
ZC = RPT // CH
ZREM = RPT - ZC * CH


def _splat(wvec, k):
    return lax.gather(
        wvec, jnp.full((L, 1), k, jnp.int32),
        lax.GatherDimensionNumbers(
            offset_dims=(), collapsed_slice_dims=(0,), start_index_map=(0,)),
        slice_sizes=(1,),
        mode=lax.GatherScatterMode.PROMISE_IN_BOUNDS)


def _sc_spmm(x_hbm, src_hbm, dst_hbm, w_hbm, out_hbm,
             agg, dst_all, w_all,
             rows0, rows1, sv0, sv1, gsem, gsem2, ssem, isem0, isem1):
    rows = (rows0, rows1)
    svs = (sv0, sv1)
    isems = (isem0, isem1)
    c = lax.axis_index("c")
    s = lax.axis_index("s")
    wid = c * NS + s
    row0 = s * RPT
    ebase = wid * EPT

    dcp = pltpu.make_async_copy(dst_hbm.at[pl.ds(ebase, EPT)], dst_all, isem0)
    wcp = pltpu.make_async_copy(w_hbm.at[pl.ds(ebase, EPT)], w_all, isem1)
    dcp.start()
    wcp.start()
    dcp.wait()
    wcp.wait()

    def _zrow(r, carry):
        for j in range(D // L):
            rows0[r, pl.ds(j * L, L)] = jnp.zeros((L,), jnp.float32)
        return carry
    lax.fori_loop(0, CH, _zrow, 0)
    for k in range(ZC):
        pltpu.sync_copy(rows0, agg.at[pl.ds(row0 + k * CH, CH)])
    pltpu.sync_copy(rows0.at[pl.ds(0, ZREM)],
                    agg.at[pl.ds(row0 + ZC * CH, ZREM)])
    plsc.subcore_barrier()

    def _src_dma(b, g):
        return pltpu.make_async_copy(
            src_hbm.at[pl.ds(ebase + g * CH, CH)], svs[b], isems[b])

    H = CH // 2

    def _gather_lo(b):
        return pltpu.make_async_copy(
            x_hbm.at[svs[b].at[pl.ds(0, H)]], rows[b].at[pl.ds(0, H)], gsem)

    def _gather_hi(b):
        return pltpu.make_async_copy(
            x_hbm.at[svs[b].at[pl.ds(H, H)]], rows[b].at[pl.ds(H, H)], gsem2)

    def _gather_start(b):
        _gather_lo(b).start()
        _gather_hi(b).start()

    def _gather_wait(b):
        _gather_lo(b).wait()
        _gather_hi(b).wait()

    def _scatter(b, off):
        return pltpu.make_async_copy(
            rows[b], agg.at[dst_all.at[pl.ds(off, CH)]], ssem)

    def _scale(b, off):
        def _blk(blk, carry):
            wvec = w_all[pl.ds(off + blk * L, L)]
            for k in range(L):
                wsplat = _splat(wvec, k)
                e = blk * L + k
                for j in range(D // L):
                    sl = pl.ds(j * L, L)
                    rows[b][e, sl] = rows[b][e, sl] * wsplat
            return carry
        lax.fori_loop(0, CH // L, _blk, 0)

    _src_dma(0, 0).start()
    _src_dma(0, 0).wait()
    _gather_start(0)
    _src_dma(1, 1).start()

    def _slot_mid(g, b):
        b1 = 1 - b
        _gather_wait(b)
        _src_dma(b, g + 2).start()
        _scatter(b1, (g - 1) * CH).wait()
        _src_dma(b1, g + 1).wait()
        _gather_start(b1)
        _scale(b, g * CH)
        _scatter(b, g * CH).start(add=True)

    _gather_wait(0)
    _src_dma(0, 2).start()
    _src_dma(1, 1).wait()
    _gather_start(1)
    _scale(0, 0)
    _scatter(0, 0).start(add=True)

    def _outer(t, carry):
        g = 1 + 2 * t
        _slot_mid(g, 1)
        _slot_mid(g + 1, 0)
        return carry
    lax.fori_loop(0, (NSLOT - 3) // 2, _outer, 0)

    _gather_wait(1)
    _scatter(0, (NSLOT - 3) * CH).wait()
    _src_dma(0, NSLOT - 1).wait()
    _gather_start(0)
    _scale(1, (NSLOT - 2) * CH)
    _scatter(1, (NSLOT - 2) * CH).start(add=True)

    _gather_wait(0)
    _scatter(1, (NSLOT - 2) * CH).wait()
    _scale(0, (NSLOT - 1) * CH)
    _scatter(0, (NSLOT - 1) * CH).start(add=True)

    _scatter((NSLOT - 1) % 2, (NSLOT - 1) * CH).wait()
    plsc.subcore_barrier()
    last = N - (NS - 1) * RPT

    @pl.when(s < NS - 1)
    def _():
        pltpu.sync_copy(agg.at[pl.ds(row0, RPT)],
                        out_hbm.at[pl.ds(c * N + row0, RPT)])

    @pl.when(s == NS - 1)
    def _():
        pltpu.sync_copy(agg.at[pl.ds((NS - 1) * RPT, last)],
                        out_hbm.at[pl.ds(c * N + (NS - 1) * RPT, last)])


_sc_call = pl.kernel(
    _sc_spmm,
    out_type=jax.ShapeDtypeStruct((2 * N, D), jnp.float32),
    mesh=plsc.VectorSubcoreMesh(core_axis_name="c", subcore_axis_name="s",
                                num_cores=NC, num_subcores=NS),
    scratch_types=[
        pltpu.VMEM_SHARED((NPAD, D), jnp.float32),
        pltpu.VMEM((EPT,), jnp.int32),
        pltpu.VMEM((EPT,), jnp.float32),
        pltpu.VMEM((CH, D), jnp.float32),
        pltpu.VMEM((CH, D), jnp.float32),
        pltpu.VMEM((CH,), jnp.int32),
        pltpu.VMEM((CH,), jnp.int32),
        pltpu.SemaphoreType.DMA,
        pltpu.SemaphoreType.DMA,
        pltpu.SemaphoreType.DMA,
        pltpu.SemaphoreType.DMA,
        pltpu.SemaphoreType.DMA,
    ],
)

BR = 1000


def _tc_body(x_ref, p0_ref, p1_ref, w_ref, b_ref, o_ref):
    sm = p0_ref[...] + p1_ref[...]
    o_ref[...] = x_ref[...] + b_ref[...] + jnp.dot(
        sm, w_ref[...], preferred_element_type=jnp.float32,
        precision=lax.Precision.HIGHEST)


def _tc_call(x, p0, p1, W, b2):
    return pl.pallas_call(
        _tc_body,
        grid=(N // BR,),
        in_specs=[
            pl.BlockSpec((BR, D), lambda i: (i, 0)),
            pl.BlockSpec((BR, D), lambda i: (i, 0)),
            pl.BlockSpec((BR, D), lambda i: (i, 0)),
            pl.BlockSpec((D, D), lambda i: (0, 0)),
            pl.BlockSpec((1, D), lambda i: (0, 0)),
        ],
        out_specs=pl.BlockSpec((BR, D), lambda i: (i, 0)),
        out_shape=jax.ShapeDtypeStruct((N, D), jnp.float32),
    )(x, p0, p1, W, b2)


def kernel(x, edge_index, edge_weight, W, b):
    dst = edge_index[0]
    src = edge_index[1]
    parts = _sc_call(x, src, dst, edge_weight)
    embed = _tc_call(x, parts[:N], parts[N:], W, b.reshape(1, D))
    return (embed, 0)


# speedup vs baseline: 1.0477x; 1.0351x over previous
import jax
import jax.numpy as jnp
from jax import lax
from jax.experimental import pallas as pl
from jax.experimental.pallas import tpu as pltpu
from jax.experimental.pallas import tpu_sc as plsc

N = 10000
D = 128
E = 320000
NC = 2
NS = 16
L = 16
NW = NC * NS
CH = 80
EPT = E // NW
NSLOT = EPT // CH
NPAD = 10112
RPT = NPAD // NS
ZC = RPT // CH
ZREM = RPT - ZC * CH


def _splat(wvec, k):
    return lax.gather(
        wvec, jnp.full((L, 1), k, jnp.int32),
        lax.GatherDimensionNumbers(
            offset_dims=(), collapsed_slice_dims=(0,), start_index_map=(0,)),
        slice_sizes=(1,),
        mode=lax.GatherScatterMode.PROMISE_IN_BOUNDS)


def _sc_spmm(x_hbm, src_hbm, dst_hbm, w_hbm, out_hbm,
             agg, dst_all, w_all,
             rows0, rows1, sv0, sv1, gsem, gsem2, ssem, isem0, isem1):
    rows = (rows0, rows1)
    svs = (sv0, sv1)
    isems = (isem0, isem1)
    c = lax.axis_index("c")
    s = lax.axis_index("s")
    wid = c * NS + s
    row0 = s * RPT
    ebase = wid * EPT

    dcp = pltpu.make_async_copy(dst_hbm.at[pl.ds(ebase, EPT)], dst_all, isem0)
    wcp = pltpu.make_async_copy(w_hbm.at[pl.ds(ebase, EPT)], w_all, isem1)
    dcp.start()
    wcp.start()
    dcp.wait()
    wcp.wait()

    def _zrow(r, carry):
        for j in range(D // L):
            rows0[r, pl.ds(j * L, L)] = jnp.zeros((L,), jnp.float32)
        return carry
    lax.fori_loop(0, CH, _zrow, 0)
    for k in range(ZC):
        pltpu.sync_copy(rows0, agg.at[pl.ds(row0 + k * CH, CH)])
    pltpu.sync_copy(rows0.at[pl.ds(0, ZREM)],
                    agg.at[pl.ds(row0 + ZC * CH, ZREM)])
    plsc.subcore_barrier()

    def _src_dma(b, g):
        return pltpu.make_async_copy(
            src_hbm.at[pl.ds(ebase + g * CH, CH)], svs[b], isems[b])

    H = CH // 2

    def _gather_lo(b):
        return pltpu.make_async_copy(
            x_hbm.at[svs[b].at[pl.ds(0, H)]], rows[b].at[pl.ds(0, H)], gsem)

    def _gather_hi(b):
        return pltpu.make_async_copy(
            x_hbm.at[svs[b].at[pl.ds(H, H)]], rows[b].at[pl.ds(H, H)], gsem2)

    def _gather_start(b):
        _gather_lo(b).start()
        _gather_hi(b).start()

    def _gather_wait(b):
        _gather_lo(b).wait()
        _gather_hi(b).wait()

    def _scatter(b, off):
        return pltpu.make_async_copy(
            rows[b], agg.at[dst_all.at[pl.ds(off, CH)]], ssem)

    def _scale(b, off):
        def _blk(blk, carry):
            wvec = w_all[pl.ds(off + blk * L, L)]
            for k in range(L):
                wsplat = _splat(wvec, k)
                e = blk * L + k
                for j in range(D // L):
                    sl = pl.ds(j * L, L)
                    rows[b][e, sl] = rows[b][e, sl] * wsplat
            return carry
        lax.fori_loop(0, CH // L, _blk, 0)

    _src_dma(0, 0).start()
    _src_dma(0, 0).wait()
    _gather_start(0)
    _src_dma(1, 1).start()

    def _slot_mid(g, b):
        b1 = 1 - b
        _gather_wait(b)
        _src_dma(b, g + 2).start()
        _scatter(b1, (g - 1) * CH).wait()
        _src_dma(b1, g + 1).wait()
        _gather_start(b1)
        _scale(b, g * CH)
        _scatter(b, g * CH).start(add=True)

    _gather_wait(0)
    _src_dma(0, 2).start()
    _src_dma(1, 1).wait()
    _gather_start(1)
    _scale(0, 0)
    _scatter(0, 0).start(add=True)

    def _outer(t, carry):
        g = 1 + 2 * t
        _slot_mid(g, 1)
        _slot_mid(g + 1, 0)
        return carry
    lax.fori_loop(0, (NSLOT - 3) // 2, _outer, 0)

    _gather_wait(1)
    _scatter(0, (NSLOT - 3) * CH).wait()
    _src_dma(0, NSLOT - 1).wait()
    _gather_start(0)
    _scale(1, (NSLOT - 2) * CH)
    _scatter(1, (NSLOT - 2) * CH).start(add=True)

    _gather_wait(0)
    _scatter(1, (NSLOT - 2) * CH).wait()
    _scale(0, (NSLOT - 1) * CH)
    _scatter(0, (NSLOT - 1) * CH).start(add=True)

    _scatter((NSLOT - 1) % 2, (NSLOT - 1) * CH).wait()
    plsc.subcore_barrier()
    last = N - (NS - 1) * RPT

    @pl.when(s < NS - 1)
    def _():
        pltpu.sync_copy(agg.at[pl.ds(row0, RPT)],
                        out_hbm.at[pl.ds(c * N + row0, RPT)])

    @pl.when(s == NS - 1)
    def _():
        pltpu.sync_copy(agg.at[pl.ds((NS - 1) * RPT, last)],
                        out_hbm.at[pl.ds(c * N + (NS - 1) * RPT, last)])


_sc_call = pl.kernel(
    _sc_spmm,
    out_type=jax.ShapeDtypeStruct((2 * N, D), jnp.float32),
    mesh=plsc.VectorSubcoreMesh(core_axis_name="c", subcore_axis_name="s",
                                num_cores=NC, num_subcores=NS),
    scratch_types=[
        pltpu.VMEM_SHARED((NPAD, D), jnp.float32),
        pltpu.VMEM((EPT,), jnp.int32),
        pltpu.VMEM((EPT,), jnp.float32),
        pltpu.VMEM((CH, D), jnp.float32),
        pltpu.VMEM((CH, D), jnp.float32),
        pltpu.VMEM((CH,), jnp.int32),
        pltpu.VMEM((CH,), jnp.int32),
        pltpu.SemaphoreType.DMA,
        pltpu.SemaphoreType.DMA,
        pltpu.SemaphoreType.DMA,
        pltpu.SemaphoreType.DMA,
        pltpu.SemaphoreType.DMA,
    ],
)

BR = 1000


def _tc_body(x_ref, p0_ref, p1_ref, w_ref, b_ref, o_ref):
    sm = p0_ref[...] + p1_ref[...]
    o_ref[...] = x_ref[...] + b_ref[...] + jnp.dot(
        sm, w_ref[...], preferred_element_type=jnp.float32,
        precision=lax.Precision.HIGHEST)


def _tc_call(x, parts, W, b2):
    return pl.pallas_call(
        _tc_body,
        grid=(N // BR,),
        in_specs=[
            pl.BlockSpec((BR, D), lambda i: (i, 0)),
            pl.BlockSpec((BR, D), lambda i: (i, 0)),
            pl.BlockSpec((BR, D), lambda i: (i + N // BR, 0)),
            pl.BlockSpec((D, D), lambda i: (0, 0)),
            pl.BlockSpec((1, D), lambda i: (0, 0)),
        ],
        out_specs=pl.BlockSpec((BR, D), lambda i: (i, 0)),
        out_shape=jax.ShapeDtypeStruct((N, D), jnp.float32),
    )(x, parts, parts, W, b2)


def kernel(x, edge_index, edge_weight, W, b):
    dst = edge_index[0]
    src = edge_index[1]
    parts = _sc_call(x, src, dst, edge_weight)
    embed = _tc_call(x, parts, W, b.reshape(1, D))
    return (embed, 0)


# speedup vs baseline: 1.0619x; 1.0136x over previous
import jax
import jax.numpy as jnp
from jax import lax
from jax.experimental import pallas as pl
from jax.experimental.pallas import tpu as pltpu
from jax.experimental.pallas import tpu_sc as plsc

N = 10000
D = 128
E = 320000
NC = 2
NS = 16
L = 16
NW = NC * NS
CH = 80
EPT = E // NW
NSLOT = EPT // CH
NPAD = 10112
RPT = NPAD // NS
ZC = RPT // CH
ZREM = RPT - ZC * CH


def _splat(wvec, k):
    return lax.gather(
        wvec, jnp.full((L, 1), k, jnp.int32),
        lax.GatherDimensionNumbers(
            offset_dims=(), collapsed_slice_dims=(0,), start_index_map=(0,)),
        slice_sizes=(1,),
        mode=lax.GatherScatterMode.PROMISE_IN_BOUNDS)


def _sc_spmm(x_hbm, src_hbm, dst_hbm, w_hbm, out_hbm,
             agg, dst_all, w_all,
             rows0, rows1, sv0, sv1, gsem, gsem2, ssem, isem0, isem1, psem):
    rows = (rows0, rows1)
    svs = (sv0, sv1)
    isems = (isem0, isem1)
    c = lax.axis_index("c")
    s = lax.axis_index("s")
    wid = c * NS + s
    row0 = s * RPT
    ebase = wid * EPT

    def _src_dma(b, g):
        return pltpu.make_async_copy(
            src_hbm.at[pl.ds(ebase + g * CH, CH)], svs[b], isems[b])

    dcp = pltpu.make_async_copy(dst_hbm.at[pl.ds(ebase, EPT)], dst_all,
                                psem)
    wcp = pltpu.make_async_copy(w_hbm.at[pl.ds(ebase, EPT)], w_all, ssem)
    dcp.start()
    wcp.start()
    _src_dma(0, 0).start()

    def _zrow(r, carry):
        for j in range(D // L):
            rows1[r, pl.ds(j * L, L)] = jnp.zeros((L,), jnp.float32)
        return carry
    lax.fori_loop(0, CH, _zrow, 0)

    H = CH // 2

    def _gather_lo(b):
        return pltpu.make_async_copy(
            x_hbm.at[svs[b].at[pl.ds(0, H)]], rows[b].at[pl.ds(0, H)], gsem)

    def _gather_hi(b):
        return pltpu.make_async_copy(
            x_hbm.at[svs[b].at[pl.ds(H, H)]], rows[b].at[pl.ds(H, H)], gsem2)

    def _gather_start(b):
        _gather_lo(b).start()
        _gather_hi(b).start()

    def _gather_wait(b):
        _gather_lo(b).wait()
        _gather_hi(b).wait()

    def _scatter(b, off):
        return pltpu.make_async_copy(
            rows[b], agg.at[dst_all.at[pl.ds(off, CH)]], ssem)

    def _scale(b, off):
        def _blk(blk, carry):
            wvec = w_all[pl.ds(off + blk * L, L)]
            for k in range(L):
                wsplat = _splat(wvec, k)
                e = blk * L + k
                for j in range(D // L):
                    sl = pl.ds(j * L, L)
                    rows[b][e, sl] = rows[b][e, sl] * wsplat
            return carry
        lax.fori_loop(0, CH // L, _blk, 0)

    _src_dma(0, 0).wait()
    _gather_start(0)
    _src_dma(1, 1).start()
    for k in range(ZC):
        pltpu.sync_copy(rows1, agg.at[pl.ds(row0 + k * CH, CH)])
    pltpu.sync_copy(rows1.at[pl.ds(0, ZREM)],
                    agg.at[pl.ds(row0 + ZC * CH, ZREM)])
    dcp.wait()
    wcp.wait()
    plsc.subcore_barrier()

    def _slot_mid(g, b):
        b1 = 1 - b
        _gather_wait(b)
        _src_dma(b, g + 2).start()
        _scatter(b1, (g - 1) * CH).wait()
        _src_dma(b1, g + 1).wait()
        _gather_start(b1)
        _scale(b, g * CH)
        _scatter(b, g * CH).start(add=True)

    _gather_wait(0)
    _src_dma(0, 2).start()
    _src_dma(1, 1).wait()
    _gather_start(1)
    _scale(0, 0)
    _scatter(0, 0).start(add=True)

    def _outer(t, carry):
        g = 1 + 2 * t
        _slot_mid(g, 1)
        _slot_mid(g + 1, 0)
        return carry
    lax.fori_loop(0, (NSLOT - 3) // 2, _outer, 0)

    _gather_wait(1)
    _scatter(0, (NSLOT - 3) * CH).wait()
    _src_dma(0, NSLOT - 1).wait()
    _gather_start(0)
    _scale(1, (NSLOT - 2) * CH)
    _scatter(1, (NSLOT - 2) * CH).start(add=True)

    _gather_wait(0)
    _scatter(1, (NSLOT - 2) * CH).wait()
    _scale(0, (NSLOT - 1) * CH)
    _scatter(0, (NSLOT - 1) * CH).start(add=True)

    _scatter((NSLOT - 1) % 2, (NSLOT - 1) * CH).wait()
    plsc.subcore_barrier()
    last = N - (NS - 1) * RPT

    @pl.when(s < NS - 1)
    def _():
        pltpu.sync_copy(agg.at[pl.ds(row0, RPT)],
                        out_hbm.at[pl.ds(c * N + row0, RPT)])

    @pl.when(s == NS - 1)
    def _():
        pltpu.sync_copy(agg.at[pl.ds((NS - 1) * RPT, last)],
                        out_hbm.at[pl.ds(c * N + (NS - 1) * RPT, last)])


_sc_call = pl.kernel(
    _sc_spmm,
    out_type=jax.ShapeDtypeStruct((2 * N, D), jnp.float32),
    mesh=plsc.VectorSubcoreMesh(core_axis_name="c", subcore_axis_name="s",
                                num_cores=NC, num_subcores=NS),
    scratch_types=[
        pltpu.VMEM_SHARED((NPAD, D), jnp.float32),
        pltpu.VMEM((EPT,), jnp.int32),
        pltpu.VMEM((EPT,), jnp.float32),
        pltpu.VMEM((CH, D), jnp.float32),
        pltpu.VMEM((CH, D), jnp.float32),
        pltpu.VMEM((CH,), jnp.int32),
        pltpu.VMEM((CH,), jnp.int32),
        pltpu.SemaphoreType.DMA,
        pltpu.SemaphoreType.DMA,
        pltpu.SemaphoreType.DMA,
        pltpu.SemaphoreType.DMA,
        pltpu.SemaphoreType.DMA,
        pltpu.SemaphoreType.DMA,
    ],
)

BR = 1000


def _tc_body(x_ref, p0_ref, p1_ref, w_ref, b_ref, o_ref):
    sm = p0_ref[...] + p1_ref[...]
    o_ref[...] = x_ref[...] + b_ref[...] + jnp.dot(
        sm, w_ref[...], preferred_element_type=jnp.float32,
        precision=lax.Precision.HIGHEST)


def _tc_call(x, parts, W, b2):
    return pl.pallas_call(
        _tc_body,
        grid=(N // BR,),
        in_specs=[
            pl.BlockSpec((BR, D), lambda i: (i, 0)),
            pl.BlockSpec((BR, D), lambda i: (i, 0)),
            pl.BlockSpec((BR, D), lambda i: (i + N // BR, 0)),
            pl.BlockSpec((D, D), lambda i: (0, 0)),
            pl.BlockSpec((1, D), lambda i: (0, 0)),
        ],
        out_specs=pl.BlockSpec((BR, D), lambda i: (i, 0)),
        out_shape=jax.ShapeDtypeStruct((N, D), jnp.float32),
    )(x, parts, parts, W, b2)


def kernel(x, edge_index, edge_weight, W, b):
    dst = edge_index[0]
    src = edge_index[1]
    parts = _sc_call(x, src, dst, edge_weight)
    embed = _tc_call(x, parts, W, b.reshape(1, D))
    return (embed, 0)
